# Initial kernel scaffold; baseline (speedup 1.0000x reference)
import jax
import jax.numpy as jnp
from jax.experimental import pallas as pl


def kernel(qubits, prev_core_allocs, current_core_allocs, core_capacities, core_connectivity, circuit_emb, key_W, key_b, key_g, key_beta, query_W, query_b, query_g, query_beta):
    raise NotImplementedError("write your pallas kernel here")


# speedup vs baseline: 3.1019x; 3.1019x over previous
import functools

import jax
import jax.numpy as jnp
from jax.experimental import pallas as pl

_BB = 8


def _ln_scale(h, g, beta, axis):
    mu = jnp.mean(h, axis=axis, keepdims=True)
    var = jnp.mean((h - mu) ** 2, axis=axis, keepdims=True)
    return (h - mu) * jax.lax.rsqrt(var + 1e-5) * g + beta


def _body(qubits_ref, prev_ref, curr_ref, caps_ref, conn_ref, ce_ref,
          kW_ref, kb_ref, kg_ref, kbeta_ref,
          qW_ref, qb_ref, qg_ref, qbeta_ref, out_ref):
    BB, Cn, Qn = out_ref.shape
    Hn = kW_ref.shape[1]

    qb = qubits_ref[...]
    q0 = qb[:, 0:1]
    q1r = qb[:, 1:2]
    double = q1r != -1
    q1 = jnp.where(double, q1r, 0)

    q_iota = jax.lax.broadcasted_iota(jnp.int32, (BB, Qn), 1)
    oh0 = (q_iota == q0).astype(jnp.float32)
    oh1 = (q_iota == q1).astype(jnp.float32)
    qm = jnp.maximum(oh0, oh1 * double.astype(jnp.float32))

    prev = prev_ref[...]
    curr = curr_ref[...]
    has_prev = jnp.any(prev != Cn, axis=1, keepdims=True)

    c_iota = jax.lax.broadcasted_iota(jnp.int32, (BB, Cn, Qn), 1)
    prev_oh = (prev[:, None, :] == c_iota).astype(jnp.float32)
    prev_core = jnp.where(has_prev[:, :, None], prev_oh, 1.0)
    curr_oh = (curr[:, None, :] == c_iota).astype(jnp.float32)

    caps = 1.0 / (caps_ref[...] + 1.0)

    prevf = prev.astype(jnp.float32)
    p0 = jnp.sum(prevf * oh0, axis=1, keepdims=True).astype(jnp.int32)
    p1 = jnp.sum(prevf * oh1, axis=1, keepdims=True).astype(jnp.int32)
    p0 = jnp.minimum(p0, Cn - 1)
    p1 = jnp.minimum(p1, Cn - 1)
    cb_iota = jax.lax.broadcasted_iota(jnp.int32, (BB, Cn), 1)
    ohp0 = (cb_iota == p0).astype(jnp.float32)
    ohp1 = (cb_iota == p1).astype(jnp.float32)
    conn = conn_ref[...]
    cost0 = jnp.dot(ohp0, conn, preferred_element_type=jnp.float32)
    cost1 = jnp.dot(ohp1, conn, preferred_element_type=jnp.float32)
    hpf = has_prev.astype(jnp.float32)
    swap = cost0 * hpf + cost1 * (hpf * double.astype(jnp.float32))
    core_cost = 1.0 / (swap + 1.0)

    aff_rows = []
    ce0_rows = []
    ce1_rows = []
    for b in range(BB):
        ce_b = ce_ref[b]
        aff_rows.append(jax.lax.dot_general(
            prev_oh[b], ce_b, (((1,), (1,)), ((), ())),
            preferred_element_type=jnp.float32))
        ce0_rows.append(jnp.dot(oh0[b:b + 1, :], ce_b,
                                preferred_element_type=jnp.float32))
        ce1_rows.append(jnp.dot(oh1[b:b + 1, :], ce_b,
                                preferred_element_type=jnp.float32))
    aff = jnp.stack(aff_rows, axis=0)
    ce_q0 = jnp.concatenate(ce0_rows, axis=0)
    ce_q1 = jnp.concatenate(ce1_rows, axis=0)
    ce_q1 = ce_q1 * double.astype(jnp.float32)

    mx = jnp.max(jnp.max(aff, axis=2), axis=1, keepdims=True)
    scale = jnp.where(mx != 0, 1.0 / jnp.where(mx != 0, mx, 1.0), 1.0)
    aff = aff * scale[:, :, None]

    feats = [qm[:, None, None, :], prev_core[:, :, None, :],
             curr_oh[:, :, None, :], caps[:, :, None, None],
             core_cost[:, :, None, None], aff[:, :, None, :],
             ce_q0[:, None, None, :], ce_q1[:, None, None, :]]

    kW = kW_ref[...]
    kb = jnp.reshape(kb_ref[...], (1, 1, Hn, 1))
    kg = jnp.reshape(kg_ref[...], (1, 1, Hn, 1))
    kbeta = jnp.reshape(kbeta_ref[...], (1, 1, Hn, 1))

    acc = jnp.broadcast_to(kb, (BB, Cn, Hn, Qn))
    for f in range(8):
        acc = acc + feats[f] * jnp.reshape(kW[f:f + 1, :], (1, 1, Hn, 1))
    key_embs = _ln_scale(jnp.maximum(acc, 0.0), kg, kbeta, axis=2)

    def at_q(oh):
        s_qm = jnp.sum(qm * oh, axis=1, keepdims=True)
        s_prev = jnp.sum(prev_core * oh[:, None, :], axis=2)
        s_curr = jnp.sum(curr_oh * oh[:, None, :], axis=2)
        s_aff = jnp.sum(aff * oh[:, None, :], axis=2)
        s_ce0 = jnp.sum(ce_q0 * oh, axis=1, keepdims=True)
        s_ce1 = jnp.sum(ce_q1 * oh, axis=1, keepdims=True)
        return [s_qm, s_prev, s_curr, caps, core_cost, s_aff, s_ce0, s_ce1]

    def embed_small(feats2, W, brow, grow, betarow):
        b3 = jnp.reshape(brow, (1, 1, Hn))
        g3 = jnp.reshape(grow, (1, 1, Hn))
        beta3 = jnp.reshape(betarow, (1, 1, Hn))
        acc2 = jnp.broadcast_to(b3, (BB, Cn, Hn))
        for f in range(8):
            ff = feats2[f]
            if ff.shape[1] == 1:
                ff = jnp.broadcast_to(ff, (BB, Cn))
            acc2 = acc2 + ff[:, :, None] * jnp.reshape(W[f:f + 1, :], (1, 1, Hn))
        return _ln_scale(jnp.maximum(acc2, 0.0), g3, beta3, axis=2)

    qW = qW_ref[...]
    q0_embs = embed_small(at_q(oh0), qW, qb_ref[...], qg_ref[...], qbeta_ref[...])
    q1_all = embed_small(at_q(oh1), kW, kb_ref[...], kg_ref[...], kbeta_ref[...])
    q1_embs = q1_all * double.astype(jnp.float32)[:, :, None]

    qsum = q0_embs + q1_embs
    out_ref[...] = jnp.sum(key_embs * qsum[:, :, :, None], axis=2)


def kernel(qubits, prev_core_allocs, current_core_allocs, core_capacities,
           core_connectivity, circuit_emb, key_W, key_b, key_g, key_beta,
           query_W, query_b, query_g, query_beta):
    B, Qn = prev_core_allocs.shape
    Cn = core_capacities.shape[1]
    Hn = key_W.shape[1]
    nb = B // _BB

    row = lambda v: jnp.reshape(v, (1, Hn))
    grid_spec = pl.GridSpec(
        grid=(nb,),
        in_specs=[
            pl.BlockSpec((_BB, 2), lambda i: (i, 0)),
            pl.BlockSpec((_BB, Qn), lambda i: (i, 0)),
            pl.BlockSpec((_BB, Qn), lambda i: (i, 0)),
            pl.BlockSpec((_BB, Cn), lambda i: (i, 0)),
            pl.BlockSpec((Cn, Cn), lambda i: (0, 0)),
            pl.BlockSpec((_BB, Qn, Qn), lambda i: (i, 0, 0)),
            pl.BlockSpec((8, Hn), lambda i: (0, 0)),
            pl.BlockSpec((1, Hn), lambda i: (0, 0)),
            pl.BlockSpec((1, Hn), lambda i: (0, 0)),
            pl.BlockSpec((1, Hn), lambda i: (0, 0)),
            pl.BlockSpec((8, Hn), lambda i: (0, 0)),
            pl.BlockSpec((1, Hn), lambda i: (0, 0)),
            pl.BlockSpec((1, Hn), lambda i: (0, 0)),
            pl.BlockSpec((1, Hn), lambda i: (0, 0)),
        ],
        out_specs=pl.BlockSpec((_BB, Cn, Qn), lambda i: (i, 0, 0)),
    )
    return pl.pallas_call(
        _body,
        grid_spec=grid_spec,
        out_shape=jax.ShapeDtypeStruct((B, Cn, Qn), jnp.float32),
    )(qubits.astype(jnp.int32), prev_core_allocs.astype(jnp.int32),
      current_core_allocs.astype(jnp.int32), core_capacities,
      core_connectivity, circuit_emb,
      key_W, row(key_b), row(key_g), row(key_beta),
      query_W, row(query_b), row(query_g), row(query_beta))


# speedup vs baseline: 4.2069x; 1.3562x over previous
import functools

import jax
import jax.numpy as jnp
from jax.experimental import pallas as pl

_BB = 8


def _ln_scale(h, g, beta, axis):
    mu = jnp.mean(h, axis=axis, keepdims=True)
    var = jnp.mean((h - mu) ** 2, axis=axis, keepdims=True)
    return (h - mu) * jax.lax.rsqrt(var + 1e-5) * g + beta


def _body(qubits_ref, prev_ref, curr_ref, caps_ref, conn_ref, ce_ref,
          kW_ref, kb_ref, kg_ref, kbeta_ref,
          qW_ref, qb_ref, qg_ref, qbeta_ref, out_ref):
    BB, Cn, Qn = out_ref.shape
    Hn = kW_ref.shape[1]

    qb = qubits_ref[...]
    q0 = qb[:, 0:1]
    q1r = qb[:, 1:2]
    double = q1r != -1
    q1 = jnp.where(double, q1r, 0)

    q_iota = jax.lax.broadcasted_iota(jnp.int32, (BB, Qn), 1)
    oh0 = (q_iota == q0).astype(jnp.float32)
    oh1 = (q_iota == q1).astype(jnp.float32)
    qm = jnp.maximum(oh0, oh1 * double.astype(jnp.float32))

    prev = prev_ref[...]
    curr = curr_ref[...]
    has_prev = jnp.any(prev != Cn, axis=1, keepdims=True)

    c_iota = jax.lax.broadcasted_iota(jnp.int32, (BB, Cn, Qn), 1)
    prev_oh = (prev[:, None, :] == c_iota).astype(jnp.float32)
    prev_core = jnp.where(has_prev[:, :, None], prev_oh, 1.0)
    curr_oh = (curr[:, None, :] == c_iota).astype(jnp.float32)

    caps = 1.0 / (caps_ref[...] + 1.0)

    prevf = prev.astype(jnp.float32)
    p0 = jnp.sum(prevf * oh0, axis=1, keepdims=True).astype(jnp.int32)
    p1 = jnp.sum(prevf * oh1, axis=1, keepdims=True).astype(jnp.int32)
    p0 = jnp.minimum(p0, Cn - 1)
    p1 = jnp.minimum(p1, Cn - 1)
    cb_iota = jax.lax.broadcasted_iota(jnp.int32, (BB, Cn), 1)
    ohp0 = (cb_iota == p0).astype(jnp.float32)
    ohp1 = (cb_iota == p1).astype(jnp.float32)
    conn = conn_ref[...]
    cost0 = jnp.dot(ohp0, conn, preferred_element_type=jnp.float32)
    cost1 = jnp.dot(ohp1, conn, preferred_element_type=jnp.float32)
    hpf = has_prev.astype(jnp.float32)
    swap = cost0 * hpf + cost1 * (hpf * double.astype(jnp.float32))
    core_cost = 1.0 / (swap + 1.0)

    aff_rows = []
    ce0_rows = []
    ce1_rows = []
    for b in range(BB):
        ce_b = ce_ref[b]
        aff_rows.append(jax.lax.dot_general(
            prev_oh[b], ce_b, (((1,), (1,)), ((), ())),
            preferred_element_type=jnp.float32))
        ce0_rows.append(jnp.dot(oh0[b:b + 1, :], ce_b,
                                preferred_element_type=jnp.float32))
        ce1_rows.append(jnp.dot(oh1[b:b + 1, :], ce_b,
                                preferred_element_type=jnp.float32))
    aff = jnp.stack(aff_rows, axis=0)
    ce_q0 = jnp.concatenate(ce0_rows, axis=0)
    ce_q1 = jnp.concatenate(ce1_rows, axis=0)
    ce_q1 = ce_q1 * double.astype(jnp.float32)

    mx = jnp.max(jnp.max(aff, axis=2), axis=1, keepdims=True)
    scale = jnp.where(mx != 0, 1.0 / jnp.where(mx != 0, mx, 1.0), 1.0)
    aff = aff * scale[:, :, None]

    kW = kW_ref[...]
    kb = jnp.reshape(kb_ref[...], (1, 1, Hn, 1))
    kg = jnp.reshape(kg_ref[...], (1, 1, Hn))
    kbeta = jnp.reshape(kbeta_ref[...], (1, 1, Hn))

    def wcol(W, f):
        return jnp.reshape(W[f:f + 1, :], (1, 1, Hn, 1))

    a_q = (kb + qm[:, None, None, :] * wcol(kW, 0)
           + ce_q0[:, None, None, :] * wcol(kW, 6)
           + ce_q1[:, None, None, :] * wcol(kW, 7))
    b_c = (caps[:, :, None, None] * wcol(kW, 3)
           + core_cost[:, :, None, None] * wcol(kW, 4))
    full = (prev_core[:, :, None, :] * wcol(kW, 1)
            + curr_oh[:, :, None, :] * wcol(kW, 2)
            + aff[:, :, None, :] * wcol(kW, 5))
    hr = jnp.maximum(full + a_q + b_c, 0.0)

    def at_q(oh):
        s_qm = jnp.sum(qm * oh, axis=1, keepdims=True)
        s_prev = jnp.sum(prev_core * oh[:, None, :], axis=2)
        s_curr = jnp.sum(curr_oh * oh[:, None, :], axis=2)
        s_aff = jnp.sum(aff * oh[:, None, :], axis=2)
        s_ce0 = jnp.sum(ce_q0 * oh, axis=1, keepdims=True)
        s_ce1 = jnp.sum(ce_q1 * oh, axis=1, keepdims=True)
        return [s_qm, s_prev, s_curr, caps, core_cost, s_aff, s_ce0, s_ce1]

    def embed_small(feats2, W, brow, grow, betarow):
        b3 = jnp.reshape(brow, (1, 1, Hn))
        g3 = jnp.reshape(grow, (1, 1, Hn))
        beta3 = jnp.reshape(betarow, (1, 1, Hn))
        acc2 = jnp.broadcast_to(b3, (BB, Cn, Hn))
        for f in range(8):
            ff = feats2[f]
            if ff.shape[1] == 1:
                ff = jnp.broadcast_to(ff, (BB, Cn))
            acc2 = acc2 + ff[:, :, None] * jnp.reshape(W[f:f + 1, :], (1, 1, Hn))
        return _ln_scale(jnp.maximum(acc2, 0.0), g3, beta3, axis=2)

    qW = qW_ref[...]
    q0_embs = embed_small(at_q(oh0), qW, qb_ref[...], qg_ref[...], qbeta_ref[...])
    q1_all = embed_small(at_q(oh1), kW, kb_ref[...], kg_ref[...], kbeta_ref[...])
    q1_embs = q1_all * double.astype(jnp.float32)[:, :, None]
    qsum = q0_embs + q1_embs

    wq = qsum * kg
    G = jnp.sum(wq, axis=2)
    Bt = jnp.sum(qsum * kbeta, axis=2)
    s0 = jnp.sum(hr, axis=2)
    s2 = jnp.sum(hr * hr, axis=2)
    sw = jnp.sum(hr * wq[:, :, :, None], axis=2)
    mu = s0 * (1.0 / Hn)
    var = s2 * (1.0 / Hn) - mu * mu
    rs = jax.lax.rsqrt(var + 1e-5)
    out_ref[...] = rs * (sw - mu * G[:, :, None]) + Bt[:, :, None]


def kernel(qubits, prev_core_allocs, current_core_allocs, core_capacities,
           core_connectivity, circuit_emb, key_W, key_b, key_g, key_beta,
           query_W, query_b, query_g, query_beta):
    B, Qn = prev_core_allocs.shape
    Cn = core_capacities.shape[1]
    Hn = key_W.shape[1]
    nb = B // _BB

    row = lambda v: jnp.reshape(v, (1, Hn))
    grid_spec = pl.GridSpec(
        grid=(nb,),
        in_specs=[
            pl.BlockSpec((_BB, 2), lambda i: (i, 0)),
            pl.BlockSpec((_BB, Qn), lambda i: (i, 0)),
            pl.BlockSpec((_BB, Qn), lambda i: (i, 0)),
            pl.BlockSpec((_BB, Cn), lambda i: (i, 0)),
            pl.BlockSpec((Cn, Cn), lambda i: (0, 0)),
            pl.BlockSpec((_BB, Qn, Qn), lambda i: (i, 0, 0)),
            pl.BlockSpec((8, Hn), lambda i: (0, 0)),
            pl.BlockSpec((1, Hn), lambda i: (0, 0)),
            pl.BlockSpec((1, Hn), lambda i: (0, 0)),
            pl.BlockSpec((1, Hn), lambda i: (0, 0)),
            pl.BlockSpec((8, Hn), lambda i: (0, 0)),
            pl.BlockSpec((1, Hn), lambda i: (0, 0)),
            pl.BlockSpec((1, Hn), lambda i: (0, 0)),
            pl.BlockSpec((1, Hn), lambda i: (0, 0)),
        ],
        out_specs=pl.BlockSpec((_BB, Cn, Qn), lambda i: (i, 0, 0)),
    )
    return pl.pallas_call(
        _body,
        grid_spec=grid_spec,
        out_shape=jax.ShapeDtypeStruct((B, Cn, Qn), jnp.float32),
    )(qubits.astype(jnp.int32), prev_core_allocs.astype(jnp.int32),
      current_core_allocs.astype(jnp.int32), core_capacities,
      core_connectivity, circuit_emb,
      key_W, row(key_b), row(key_g), row(key_beta),
      query_W, row(query_b), row(query_g), row(query_beta))


# speedup vs baseline: 4.7726x; 1.1345x over previous
import functools

import jax
import jax.numpy as jnp
from jax.experimental import pallas as pl

_BB = 16


def _ln_scale(h, g, beta, axis):
    mu = jnp.mean(h, axis=axis, keepdims=True)
    var = jnp.mean((h - mu) ** 2, axis=axis, keepdims=True)
    return (h - mu) * jax.lax.rsqrt(var + 1e-5) * g + beta


def _body(qubits_ref, prev_ref, curr_ref, caps_ref, conn_ref, ce_ref,
          kW_ref, kb_ref, kg_ref, kbeta_ref,
          qW_ref, qb_ref, qg_ref, qbeta_ref, out_ref):
    BB, Cn, Qn = out_ref.shape
    Hn = kW_ref.shape[1]

    qb = qubits_ref[...]
    q0 = qb[:, 0:1]
    q1r = qb[:, 1:2]
    double = q1r != -1
    q1 = jnp.where(double, q1r, 0)

    q_iota = jax.lax.broadcasted_iota(jnp.int32, (BB, Qn), 1)
    oh0 = (q_iota == q0).astype(jnp.float32)
    oh1 = (q_iota == q1).astype(jnp.float32)
    qm = jnp.maximum(oh0, oh1 * double.astype(jnp.float32))

    prev = prev_ref[...]
    curr = curr_ref[...]
    has_prev = jnp.any(prev != Cn, axis=1, keepdims=True)

    c_iota = jax.lax.broadcasted_iota(jnp.int32, (BB, Cn, Qn), 1)
    prev_oh = (prev[:, None, :] == c_iota).astype(jnp.float32)
    prev_core = jnp.where(has_prev[:, :, None], prev_oh, 1.0)
    curr_oh = (curr[:, None, :] == c_iota).astype(jnp.float32)

    caps = 1.0 / (caps_ref[...] + 1.0)

    prevf = prev.astype(jnp.float32)
    p0 = jnp.sum(prevf * oh0, axis=1, keepdims=True).astype(jnp.int32)
    p1 = jnp.sum(prevf * oh1, axis=1, keepdims=True).astype(jnp.int32)
    p0 = jnp.minimum(p0, Cn - 1)
    p1 = jnp.minimum(p1, Cn - 1)
    cb_iota = jax.lax.broadcasted_iota(jnp.int32, (BB, Cn), 1)
    ohp0 = (cb_iota == p0).astype(jnp.float32)
    ohp1 = (cb_iota == p1).astype(jnp.float32)
    conn = conn_ref[...]
    cost0 = jnp.dot(ohp0, conn, preferred_element_type=jnp.float32)
    cost1 = jnp.dot(ohp1, conn, preferred_element_type=jnp.float32)
    hpf = has_prev.astype(jnp.float32)
    swap = cost0 * hpf + cost1 * (hpf * double.astype(jnp.float32))
    core_cost = 1.0 / (swap + 1.0)

    aff_rows = []
    ce0_rows = []
    ce1_rows = []
    for b in range(BB):
        ce_b = ce_ref[b]
        aff_rows.append(jax.lax.dot_general(
            prev_oh[b], ce_b, (((1,), (1,)), ((), ())),
            preferred_element_type=jnp.float32))
        ce0_rows.append(jnp.dot(oh0[b:b + 1, :], ce_b,
                                preferred_element_type=jnp.float32))
        ce1_rows.append(jnp.dot(oh1[b:b + 1, :], ce_b,
                                preferred_element_type=jnp.float32))
    aff = jnp.stack(aff_rows, axis=0)
    ce_q0 = jnp.concatenate(ce0_rows, axis=0)
    ce_q1 = jnp.concatenate(ce1_rows, axis=0)
    ce_q1 = ce_q1 * double.astype(jnp.float32)

    mx = jnp.max(jnp.max(aff, axis=2), axis=1, keepdims=True)
    scale = jnp.where(mx != 0, 1.0 / jnp.where(mx != 0, mx, 1.0), 1.0)
    aff = aff * scale[:, :, None]

    kW = kW_ref[...]
    kb = jnp.reshape(kb_ref[...], (1, 1, Hn, 1))
    kg = jnp.reshape(kg_ref[...], (1, 1, Hn))
    kbeta = jnp.reshape(kbeta_ref[...], (1, 1, Hn))

    def wcol(W, f):
        return jnp.reshape(W[f:f + 1, :], (1, 1, Hn, 1))

    a_q = (kb + qm[:, None, None, :] * wcol(kW, 0)
           + ce_q0[:, None, None, :] * wcol(kW, 6)
           + ce_q1[:, None, None, :] * wcol(kW, 7))
    b_c = (caps[:, :, None, None] * wcol(kW, 3)
           + core_cost[:, :, None, None] * wcol(kW, 4))
    full = (prev_core[:, :, None, :] * wcol(kW, 1)
            + curr_oh[:, :, None, :] * wcol(kW, 2)
            + aff[:, :, None, :] * wcol(kW, 5))
    hr = jnp.maximum(full + a_q + b_c, 0.0)

    def at_q(oh):
        s_qm = jnp.sum(qm * oh, axis=1, keepdims=True)
        s_prev = jnp.sum(prev_core * oh[:, None, :], axis=2)
        s_curr = jnp.sum(curr_oh * oh[:, None, :], axis=2)
        s_aff = jnp.sum(aff * oh[:, None, :], axis=2)
        s_ce0 = jnp.sum(ce_q0 * oh, axis=1, keepdims=True)
        s_ce1 = jnp.sum(ce_q1 * oh, axis=1, keepdims=True)
        return [s_qm, s_prev, s_curr, caps, core_cost, s_aff, s_ce0, s_ce1]

    def embed_small(feats2, W, brow, grow, betarow):
        b3 = jnp.reshape(brow, (1, 1, Hn))
        g3 = jnp.reshape(grow, (1, 1, Hn))
        beta3 = jnp.reshape(betarow, (1, 1, Hn))
        acc2 = jnp.broadcast_to(b3, (BB, Cn, Hn))
        for f in range(8):
            ff = feats2[f]
            if ff.shape[1] == 1:
                ff = jnp.broadcast_to(ff, (BB, Cn))
            acc2 = acc2 + ff[:, :, None] * jnp.reshape(W[f:f + 1, :], (1, 1, Hn))
        return _ln_scale(jnp.maximum(acc2, 0.0), g3, beta3, axis=2)

    qW = qW_ref[...]
    q0_embs = embed_small(at_q(oh0), qW, qb_ref[...], qg_ref[...], qbeta_ref[...])
    q1_all = embed_small(at_q(oh1), kW, kb_ref[...], kg_ref[...], kbeta_ref[...])
    q1_embs = q1_all * double.astype(jnp.float32)[:, :, None]
    qsum = q0_embs + q1_embs

    wq = qsum * kg
    G = jnp.sum(wq, axis=2)
    Bt = jnp.sum(qsum * kbeta, axis=2)
    s0 = jnp.sum(hr, axis=2)
    s2 = jnp.sum(hr * hr, axis=2)
    sw = jnp.sum(hr * wq[:, :, :, None], axis=2)
    mu = s0 * (1.0 / Hn)
    var = s2 * (1.0 / Hn) - mu * mu
    rs = jax.lax.rsqrt(var + 1e-5)
    out_ref[...] = rs * (sw - mu * G[:, :, None]) + Bt[:, :, None]


def kernel(qubits, prev_core_allocs, current_core_allocs, core_capacities,
           core_connectivity, circuit_emb, key_W, key_b, key_g, key_beta,
           query_W, query_b, query_g, query_beta):
    B, Qn = prev_core_allocs.shape
    Cn = core_capacities.shape[1]
    Hn = key_W.shape[1]
    nb = B // _BB

    row = lambda v: jnp.reshape(v, (1, Hn))
    grid_spec = pl.GridSpec(
        grid=(nb,),
        in_specs=[
            pl.BlockSpec((_BB, 2), lambda i: (i, 0)),
            pl.BlockSpec((_BB, Qn), lambda i: (i, 0)),
            pl.BlockSpec((_BB, Qn), lambda i: (i, 0)),
            pl.BlockSpec((_BB, Cn), lambda i: (i, 0)),
            pl.BlockSpec((Cn, Cn), lambda i: (0, 0)),
            pl.BlockSpec((_BB, Qn, Qn), lambda i: (i, 0, 0)),
            pl.BlockSpec((8, Hn), lambda i: (0, 0)),
            pl.BlockSpec((1, Hn), lambda i: (0, 0)),
            pl.BlockSpec((1, Hn), lambda i: (0, 0)),
            pl.BlockSpec((1, Hn), lambda i: (0, 0)),
            pl.BlockSpec((8, Hn), lambda i: (0, 0)),
            pl.BlockSpec((1, Hn), lambda i: (0, 0)),
            pl.BlockSpec((1, Hn), lambda i: (0, 0)),
            pl.BlockSpec((1, Hn), lambda i: (0, 0)),
        ],
        out_specs=pl.BlockSpec((_BB, Cn, Qn), lambda i: (i, 0, 0)),
    )
    return pl.pallas_call(
        _body,
        grid_spec=grid_spec,
        out_shape=jax.ShapeDtypeStruct((B, Cn, Qn), jnp.float32),
    )(qubits.astype(jnp.int32), prev_core_allocs.astype(jnp.int32),
      current_core_allocs.astype(jnp.int32), core_capacities,
      core_connectivity, circuit_emb,
      key_W, row(key_b), row(key_g), row(key_beta),
      query_W, row(query_b), row(query_g), row(query_beta))
